# baseline (speedup 1.0000x reference)
import functools

import jax
import jax.numpy as jnp
from jax import lax
from jax.experimental import pallas as pl
from jax.experimental.pallas import tpu as pltpu
from jax.experimental.pallas import tpu_sc as plsc

N = 10000
D = 128
E = 320000
G = 64
L = 3

NC = 2
NS = 16
NW = NC * NS
CHUNK = 128
NCH = 80
EPW = NCH * CHUNK
EPAD = NW * EPW
ROWS = 10112
RPT = ROWS // NS
RB = 1000
NBLK = N // RB
EPS = 1e-5

DW = D


@functools.cache
def _get_deg_kernel():
    mesh = plsc.VectorSubcoreMesh(core_axis_name="c", subcore_axis_name="s")

    @functools.partial(
        pl.kernel,
        out_type=jax.ShapeDtypeStruct((NC, ROWS, DW), jnp.float32),
        mesh=mesh,
        scratch_types=[
            pltpu.VMEM((NCH, CHUNK), jnp.int32),
            pltpu.VMEM((CHUNK, DW), jnp.float32),
            pltpu.VMEM_SHARED((ROWS, DW), jnp.float32),
        ],
    )
    def deg_kernel(dst_hbm, zeros_hbm, ones_hbm, out_hbm, dst_v, ones_v, acc_sh):
        c = lax.axis_index("c")
        s = lax.axis_index("s")
        wid = c * NS + s
        pltpu.sync_copy(zeros_hbm.at[pl.ds(s * RPT, RPT)],
                        acc_sh.at[pl.ds(s * RPT, RPT)])
        pltpu.sync_copy(ones_hbm, ones_v)
        pltpu.sync_copy(dst_hbm.at[wid], dst_v)
        plsc.subcore_barrier()

        def chunk_body(j, carry):
            pltpu.sync_copy(ones_v, acc_sh.at[dst_v.at[j]], add=True)
            return carry

        lax.fori_loop(0, NCH, chunk_body, 0)
        plsc.subcore_barrier()
        pltpu.sync_copy(acc_sh.at[pl.ds(s * RPT, RPT)],
                        out_hbm.at[c, pl.ds(s * RPT, RPT)])

    return deg_kernel


@functools.cache
def _get_scatter_kernel():
    mesh = plsc.VectorSubcoreMesh(core_axis_name="c", subcore_axis_name="s")

    @functools.partial(
        pl.kernel,
        out_type=jax.ShapeDtypeStruct((NC, ROWS, D), jnp.float32),
        mesh=mesh,
        scratch_types=[
            pltpu.VMEM((NCH // 2, CHUNK), jnp.int32),
            pltpu.VMEM((NCH // 2, CHUNK), jnp.int32),
            pltpu.VMEM((CHUNK, D), jnp.float32),
            pltpu.VMEM((CHUNK, D), jnp.float32),
            pltpu.SemaphoreType.DMA,
            pltpu.SemaphoreType.DMA,
            pltpu.SemaphoreType.DMA,
            pltpu.SemaphoreType.DMA,
            pltpu.VMEM_SHARED((ROWS, D), jnp.float32),
        ],
    )
    def scatter_kernel(src_hbm, dst_hbm, y_hbm, zeros_hbm, out_hbm,
                       src_v, dst_v, buf_a, buf_b, gs_a, gs_b, ss_a, ss_b,
                       acc_sh):
        c = lax.axis_index("c")
        s = lax.axis_index("s")
        wid = c * NS + s
        HCH = NCH // 2
        pltpu.sync_copy(zeros_hbm.at[pl.ds(s * RPT, RPT)],
                        acc_sh.at[pl.ds(s * RPT, RPT)])
        plsc.subcore_barrier()

        def wait_gather(j, buf, sem):
            pltpu.make_async_copy(y_hbm.at[src_v.at[j]], buf, sem).wait()

        def wait_scatter(j, buf, sem):
            pltpu.make_async_copy(buf, acc_sh.at[dst_v.at[j]], sem).wait()

        def process_half(h):
            pltpu.sync_copy(src_hbm.at[wid, pl.ds(h * HCH, HCH)], src_v)
            pltpu.sync_copy(dst_hbm.at[wid, pl.ds(h * HCH, HCH)], dst_v)
            pltpu.async_copy(y_hbm.at[src_v.at[0]], buf_a, gs_a)
            wait_gather(0, buf_a, gs_a)
            pltpu.async_copy(buf_a, acc_sh.at[dst_v.at[0]], ss_a, add=True)
            pltpu.async_copy(y_hbm.at[src_v.at[1]], buf_b, gs_b)
            wait_gather(1, buf_b, gs_b)
            pltpu.async_copy(buf_b, acc_sh.at[dst_v.at[1]], ss_b, add=True)
            wait_scatter(0, buf_a, ss_a)
            pltpu.async_copy(y_hbm.at[src_v.at[2]], buf_a, gs_a)

            def chunk_body(i, carry):
                j0 = 2 * i
                j1 = 2 * i + 1
                jn = lax.rem(j1 + 1, HCH)
                wait_gather(j0, buf_a, gs_a)
                pltpu.async_copy(buf_a, acc_sh.at[dst_v.at[j0]], ss_a,
                                 add=True)
                wait_scatter(j1 - 2, buf_b, ss_b)
                pltpu.async_copy(y_hbm.at[src_v.at[j1]], buf_b, gs_b)
                wait_gather(j1, buf_b, gs_b)
                pltpu.async_copy(buf_b, acc_sh.at[dst_v.at[j1]], ss_b,
                                 add=True)
                wait_scatter(j0, buf_a, ss_a)
                pltpu.async_copy(y_hbm.at[src_v.at[jn]], buf_a, gs_a)
                return carry

            lax.fori_loop(1, HCH // 2, chunk_body, 0)
            wait_gather(0, buf_a, gs_a)
            wait_scatter(HCH - 1, buf_b, ss_b)

        process_half(0)
        process_half(1)
        plsc.subcore_barrier()
        pltpu.sync_copy(acc_sh.at[pl.ds(s * RPT, RPT)],
                        out_hbm.at[c, pl.ds(s * RPT, RPT)])

    return scatter_kernel


def _matmul_body(x_ref, w_ref, z_ref):
    z_ref[...] = jnp.dot(x_ref[...], w_ref[...],
                         preferred_element_type=jnp.float32)


def _tc_matmul(x, w):
    return pl.pallas_call(
        _matmul_body,
        grid=(NBLK,),
        in_specs=[
            pl.BlockSpec((RB, D), lambda b: (b, 0)),
            pl.BlockSpec((D, D), lambda b: (0, 0)),
        ],
        out_specs=pl.BlockSpec((RB, D), lambda b: (b, 0)),
        out_shape=jax.ShapeDtypeStruct((N, D), jnp.float32),
    )(x, w)


def _scale_body(d_ref, z_ref, y_ref, dinv_ref):
    dinv = 1.0 / jnp.sqrt(d_ref[0, :, 0:1] + d_ref[1, :, 0:1] + 1.0)
    dinv_ref[...] = dinv
    y_ref[...] = z_ref[...] * dinv


def _tc_scale(deg_parts, z):
    return pl.pallas_call(
        _scale_body,
        grid=(NBLK,),
        in_specs=[
            pl.BlockSpec((NC, RB, DW), lambda b: (0, b, 0)),
            pl.BlockSpec((RB, D), lambda b: (b, 0)),
        ],
        out_specs=[
            pl.BlockSpec((RB, D), lambda b: (b, 0)),
            pl.BlockSpec((RB, 1), lambda b: (b, 0)),
        ],
        out_shape=[
            jax.ShapeDtypeStruct((N, D), jnp.float32),
            jax.ShapeDtypeStruct((N, 1), jnp.float32),
        ],
    )(deg_parts, z)


def _make_combine_body(has_next):
    def body(*refs):
        if has_next:
            (s_ref, y_ref, dinv_ref, bb_ref, g_ref, be_ref, bat_ref, wn_ref,
             pool_ref, ynext_ref, stat_ref, r_scr) = refs
        else:
            (s_ref, y_ref, dinv_ref, bb_ref, g_ref, be_ref, bat_ref,
             pool_ref, stat_ref, r_scr) = refs
        p = pl.program_id(0)
        blk = pl.program_id(1)

        @pl.when(p == 0)
        def _():
            r = jnp.maximum(
                dinv_ref[...] * (s_ref[0] + s_ref[1] + y_ref[...])
                + bb_ref[...],
                0.0,
            )
            r_scr[pl.ds(blk * RB, RB), :] = r
            srow = jnp.sum(r, axis=0, keepdims=True)
            s2row = jnp.sum(r * r, axis=0, keepdims=True)

            @pl.when(blk == 0)
            def _():
                stat_ref[0:1] = srow
                stat_ref[1:2] = s2row

            @pl.when(blk != 0)
            def _():
                stat_ref[0:1] += srow
                stat_ref[1:2] += s2row

        @pl.when(p == 1)
        def _():
            r = r_scr[pl.ds(blk * RB, RB), :]
            mean = stat_ref[0:1] / N
            var = stat_ref[1:2] / N - mean * mean
            hbn = (r - mean) * (g_ref[...] / jnp.sqrt(var + EPS)) + be_ref[...]
            oh = (lax.broadcasted_iota(jnp.int32, (G, RB), 0)
                  == bat_ref[0]).astype(jnp.float32)
            contr = jnp.dot(oh, hbn, preferred_element_type=jnp.float32,
                            precision=lax.Precision.HIGHEST)

            @pl.when(blk == 0)
            def _():
                pool_ref[...] = contr

            @pl.when(blk != 0)
            def _():
                pool_ref[...] += contr

            if has_next:
                ynext_ref[...] = (
                    jnp.dot(hbn, wn_ref[...], preferred_element_type=jnp.float32)
                    * dinv_ref[...]
                )

    return body


def _tc_combine(S, y, dinv_col, bb, g, be, bat_row, wn):
    has_next = wn is not None
    in_specs = [
        pl.BlockSpec((NC, RB, D), lambda p, b: (0, b * (1 - p), 0)),
        pl.BlockSpec((RB, D), lambda p, b: (b * (1 - p), 0)),
        pl.BlockSpec((RB, 1), lambda p, b: (b, 0)),
        pl.BlockSpec((1, D), lambda p, b: (0, 0)),
        pl.BlockSpec((1, D), lambda p, b: (0, 0)),
        pl.BlockSpec((1, D), lambda p, b: (0, 0)),
        pl.BlockSpec((1, 1, RB), lambda p, b: (b, 0, 0)),
    ]
    args = [S, y, dinv_col, bb, g, be, bat_row]
    out_specs = [pl.BlockSpec((G, D), lambda p, b: (0, 0))]
    out_shape = [jax.ShapeDtypeStruct((G, D), jnp.float32)]
    if has_next:
        in_specs.append(pl.BlockSpec((D, D), lambda p, b: (0, 0)))
        args.append(wn)
        out_specs.append(pl.BlockSpec((RB, D), lambda p, b: (b, 0)))
        out_shape.append(jax.ShapeDtypeStruct((N, D), jnp.float32))
    res = pl.pallas_call(
        _make_combine_body(has_next),
        grid=(2, NBLK),
        in_specs=in_specs,
        out_specs=out_specs,
        out_shape=out_shape,
        scratch_shapes=[pltpu.VMEM((2, D), jnp.float32),
                        pltpu.VMEM((N, D), jnp.float32)],
    )(*args)
    if has_next:
        return res[0], res[1]
    return res[0], None


def kernel(x, edge_index, batch, Ws, bs, gammas, betas):
    src = edge_index[0]
    dst = edge_index[1]
    pad = EPAD - E
    src_p = jnp.concatenate(
        [src, jnp.arange(pad, dtype=jnp.int32)]).reshape(NW, NCH, CHUNK)
    dst_p = jnp.concatenate(
        [dst, jnp.full((pad,), N, jnp.int32)]).reshape(NW, NCH, CHUNK)
    zeros_hbm = jnp.zeros((ROWS, D), jnp.float32)
    ones_rows = jnp.ones((CHUNK, DW), jnp.float32)

    deg_parts = _get_deg_kernel()(dst_p, zeros_hbm, ones_rows)
    z = _tc_matmul(x, Ws[0])
    y, dinv_col = _tc_scale(deg_parts, z)
    bat_row = batch.reshape(NBLK, 1, RB)
    pools = []
    for i in range(L):
        S = _get_scatter_kernel()(src_p, dst_p, y, zeros_hbm)
        wn = Ws[i + 1] if i + 1 < L else None
        pool, y = _tc_combine(
            S, y, dinv_col,
            bs[i].reshape(1, D), gammas[i].reshape(1, D),
            betas[i].reshape(1, D), bat_row, wn)
        pools.append(pool)
    return jnp.concatenate(pools, axis=1)


# speedup vs baseline: 1.0031x; 1.0031x over previous
import functools

import jax
import jax.numpy as jnp
from jax import lax
from jax.experimental import pallas as pl
from jax.experimental.pallas import tpu as pltpu
from jax.experimental.pallas import tpu_sc as plsc

N = 10000
D = 128
E = 320000
G = 64
L = 3

NC = 2
NS = 16
NW = NC * NS
CHUNK = 128
NCH = 80
EPW = NCH * CHUNK
EPAD = NW * EPW
ROWS = 10112
RPT = ROWS // NS
RB = 1000
NBLK = N // RB
EPS = 1e-5

DW = D


@functools.cache
def _get_deg_kernel():
    mesh = plsc.VectorSubcoreMesh(core_axis_name="c", subcore_axis_name="s")

    @functools.partial(
        pl.kernel,
        out_type=jax.ShapeDtypeStruct((NC, ROWS, DW), jnp.float32),
        mesh=mesh,
        scratch_types=[
            pltpu.VMEM((NCH, CHUNK), jnp.int32),
            pltpu.VMEM((CHUNK, DW), jnp.float32),
            pltpu.SemaphoreType.DMA,
            pltpu.VMEM_SHARED((ROWS, DW), jnp.float32),
        ],
    )
    def deg_kernel(dst_hbm, zeros_hbm, ones_hbm, out_hbm, dst_v, ones_v,
                   sem, acc_sh):
        c = lax.axis_index("c")
        s = lax.axis_index("s")
        wid = c * NS + s
        pltpu.sync_copy(zeros_hbm.at[pl.ds(s * RPT, RPT)],
                        acc_sh.at[pl.ds(s * RPT, RPT)])
        pltpu.sync_copy(ones_hbm, ones_v)
        pltpu.sync_copy(dst_hbm.at[wid], dst_v)
        plsc.subcore_barrier()

        def fire(j, carry):
            pltpu.async_copy(ones_v, acc_sh.at[dst_v.at[j]], sem, add=True)
            return carry

        def drain(j, carry):
            pltpu.make_async_copy(ones_v, acc_sh.at[dst_v.at[j]], sem).wait()
            return carry

        lax.fori_loop(0, NCH, fire, 0)
        lax.fori_loop(0, NCH, drain, 0)
        plsc.subcore_barrier()
        pltpu.sync_copy(acc_sh.at[pl.ds(s * RPT, RPT)],
                        out_hbm.at[c, pl.ds(s * RPT, RPT)])

    return deg_kernel


@functools.cache
def _get_scatter_kernel():
    mesh = plsc.VectorSubcoreMesh(core_axis_name="c", subcore_axis_name="s")

    @functools.partial(
        pl.kernel,
        out_type=jax.ShapeDtypeStruct((NC, ROWS, D), jnp.float32),
        mesh=mesh,
        scratch_types=[
            pltpu.VMEM((NCH // 2, CHUNK), jnp.int32),
            pltpu.VMEM((NCH // 2, CHUNK), jnp.int32),
            pltpu.VMEM((CHUNK, D), jnp.float32),
            pltpu.VMEM((CHUNK, D), jnp.float32),
            pltpu.SemaphoreType.DMA,
            pltpu.SemaphoreType.DMA,
            pltpu.VMEM_SHARED((ROWS, D), jnp.float32),
        ],
    )
    def scatter_kernel(src_hbm, dst_hbm, y_hbm, zeros_hbm, out_hbm,
                       src_v, dst_v, buf_a, buf_b, sem_a, sem_b, acc_sh):
        c = lax.axis_index("c")
        s = lax.axis_index("s")
        wid = c * NS + s
        HCH = NCH // 2
        pltpu.sync_copy(zeros_hbm.at[pl.ds(s * RPT, RPT)],
                        acc_sh.at[pl.ds(s * RPT, RPT)])
        plsc.subcore_barrier()

        def process_half(h):
            pltpu.sync_copy(src_hbm.at[wid, pl.ds(h * HCH, HCH)], src_v)
            pltpu.sync_copy(dst_hbm.at[wid, pl.ds(h * HCH, HCH)], dst_v)
            pltpu.async_copy(y_hbm.at[src_v.at[0]], buf_a, sem_a)

            def chunk_body(i, carry):
                j0 = 2 * i
                j1 = 2 * i + 1
                jn = lax.rem(j1 + 1, HCH)
                pltpu.make_async_copy(
                    y_hbm.at[src_v.at[j0]], buf_a, sem_a).wait()
                pltpu.async_copy(y_hbm.at[src_v.at[j1]], buf_b, sem_b)
                pltpu.sync_copy(buf_a, acc_sh.at[dst_v.at[j0]], add=True)
                pltpu.make_async_copy(
                    y_hbm.at[src_v.at[j1]], buf_b, sem_b).wait()
                pltpu.async_copy(y_hbm.at[src_v.at[jn]], buf_a, sem_a)
                pltpu.sync_copy(buf_b, acc_sh.at[dst_v.at[j1]], add=True)
                return carry

            lax.fori_loop(0, HCH // 2, chunk_body, 0)
            pltpu.make_async_copy(y_hbm.at[src_v.at[0]], buf_a, sem_a).wait()

        process_half(0)
        process_half(1)
        plsc.subcore_barrier()
        pltpu.sync_copy(acc_sh.at[pl.ds(s * RPT, RPT)],
                        out_hbm.at[c, pl.ds(s * RPT, RPT)])

    return scatter_kernel


def _matmul_body(x_ref, w_ref, z_ref):
    z_ref[...] = jnp.dot(x_ref[...], w_ref[...],
                         preferred_element_type=jnp.float32)


def _tc_matmul(x, w):
    return pl.pallas_call(
        _matmul_body,
        grid=(NBLK,),
        in_specs=[
            pl.BlockSpec((RB, D), lambda b: (b, 0)),
            pl.BlockSpec((D, D), lambda b: (0, 0)),
        ],
        out_specs=pl.BlockSpec((RB, D), lambda b: (b, 0)),
        out_shape=jax.ShapeDtypeStruct((N, D), jnp.float32),
    )(x, w)


def _scale_body(d_ref, z_ref, y_ref, dinv_ref):
    dinv = 1.0 / jnp.sqrt(d_ref[0, :, 0:1] + d_ref[1, :, 0:1] + 1.0)
    dinv_ref[...] = dinv
    y_ref[...] = z_ref[...] * dinv


def _tc_scale(deg_parts, z):
    return pl.pallas_call(
        _scale_body,
        grid=(NBLK,),
        in_specs=[
            pl.BlockSpec((NC, RB, DW), lambda b: (0, b, 0)),
            pl.BlockSpec((RB, D), lambda b: (b, 0)),
        ],
        out_specs=[
            pl.BlockSpec((RB, D), lambda b: (b, 0)),
            pl.BlockSpec((RB, 1), lambda b: (b, 0)),
        ],
        out_shape=[
            jax.ShapeDtypeStruct((N, D), jnp.float32),
            jax.ShapeDtypeStruct((N, 1), jnp.float32),
        ],
    )(deg_parts, z)


def _make_combine_body(has_next):
    def body(*refs):
        if has_next:
            (s_ref, y_ref, dinv_ref, bb_ref, g_ref, be_ref, bat_ref, wn_ref,
             pool_ref, ynext_ref, stat_ref, r_scr) = refs
        else:
            (s_ref, y_ref, dinv_ref, bb_ref, g_ref, be_ref, bat_ref,
             pool_ref, stat_ref, r_scr) = refs
        p = pl.program_id(0)
        blk = pl.program_id(1)

        @pl.when(p == 0)
        def _():
            r = jnp.maximum(
                dinv_ref[...] * (s_ref[0] + s_ref[1] + y_ref[...])
                + bb_ref[...],
                0.0,
            )
            r_scr[pl.ds(blk * RB, RB), :] = r
            srow = jnp.sum(r, axis=0, keepdims=True)
            s2row = jnp.sum(r * r, axis=0, keepdims=True)

            @pl.when(blk == 0)
            def _():
                stat_ref[0:1] = srow
                stat_ref[1:2] = s2row

            @pl.when(blk != 0)
            def _():
                stat_ref[0:1] += srow
                stat_ref[1:2] += s2row

        @pl.when(p == 1)
        def _():
            r = r_scr[pl.ds(blk * RB, RB), :]
            mean = stat_ref[0:1] / N
            var = stat_ref[1:2] / N - mean * mean
            hbn = (r - mean) * (g_ref[...] / jnp.sqrt(var + EPS)) + be_ref[...]
            oh = (lax.broadcasted_iota(jnp.int32, (G, RB), 0)
                  == bat_ref[0]).astype(jnp.float32)
            contr = jnp.dot(oh, hbn, preferred_element_type=jnp.float32,
                            precision=lax.Precision.HIGHEST)

            @pl.when(blk == 0)
            def _():
                pool_ref[...] = contr

            @pl.when(blk != 0)
            def _():
                pool_ref[...] += contr

            if has_next:
                ynext_ref[...] = (
                    jnp.dot(hbn, wn_ref[...], preferred_element_type=jnp.float32)
                    * dinv_ref[...]
                )

    return body


def _tc_combine(S, y, dinv_col, bb, g, be, bat_row, wn):
    has_next = wn is not None
    in_specs = [
        pl.BlockSpec((NC, RB, D), lambda p, b: (0, b * (1 - p), 0)),
        pl.BlockSpec((RB, D), lambda p, b: (b * (1 - p), 0)),
        pl.BlockSpec((RB, 1), lambda p, b: (b, 0)),
        pl.BlockSpec((1, D), lambda p, b: (0, 0)),
        pl.BlockSpec((1, D), lambda p, b: (0, 0)),
        pl.BlockSpec((1, D), lambda p, b: (0, 0)),
        pl.BlockSpec((1, 1, RB), lambda p, b: (b, 0, 0)),
    ]
    args = [S, y, dinv_col, bb, g, be, bat_row]
    out_specs = [pl.BlockSpec((G, D), lambda p, b: (0, 0))]
    out_shape = [jax.ShapeDtypeStruct((G, D), jnp.float32)]
    if has_next:
        in_specs.append(pl.BlockSpec((D, D), lambda p, b: (0, 0)))
        args.append(wn)
        out_specs.append(pl.BlockSpec((RB, D), lambda p, b: (b, 0)))
        out_shape.append(jax.ShapeDtypeStruct((N, D), jnp.float32))
    res = pl.pallas_call(
        _make_combine_body(has_next),
        grid=(2, NBLK),
        in_specs=in_specs,
        out_specs=out_specs,
        out_shape=out_shape,
        scratch_shapes=[pltpu.VMEM((2, D), jnp.float32),
                        pltpu.VMEM((N, D), jnp.float32)],
    )(*args)
    if has_next:
        return res[0], res[1]
    return res[0], None


def kernel(x, edge_index, batch, Ws, bs, gammas, betas):
    src = edge_index[0]
    dst = edge_index[1]
    pad = EPAD - E
    src_p = jnp.concatenate(
        [src, jnp.arange(pad, dtype=jnp.int32)]).reshape(NW, NCH, CHUNK)
    dst_p = jnp.concatenate(
        [dst, jnp.full((pad,), N, jnp.int32)]).reshape(NW, NCH, CHUNK)
    zeros_hbm = jnp.zeros((ROWS, D), jnp.float32)
    ones_rows = jnp.ones((CHUNK, DW), jnp.float32)

    deg_parts = _get_deg_kernel()(dst_p, zeros_hbm, ones_rows)
    z = _tc_matmul(x, Ws[0])
    y, dinv_col = _tc_scale(deg_parts, z)
    bat_row = batch.reshape(NBLK, 1, RB)
    pools = []
    for i in range(L):
        S = _get_scatter_kernel()(src_p, dst_p, y, zeros_hbm)
        wn = Ws[i + 1] if i + 1 < L else None
        pool, y = _tc_combine(
            S, y, dinv_col,
            bs[i].reshape(1, D), gammas[i].reshape(1, D),
            betas[i].reshape(1, D), bat_row, wn)
        pools.append(pool)
    return jnp.concatenate(pools, axis=1)
